# Initial kernel scaffold; baseline (speedup 1.0000x reference)
import jax
import jax.numpy as jnp
from jax.experimental import pallas as pl


def kernel(x_molecule, x_reaction, edge_index_m2r, edge_index_r2m, params):
    raise NotImplementedError("write your pallas kernel here")


# speedup vs baseline: 3.9120x; 3.9120x over previous
import functools

import jax
import jax.numpy as jnp
from jax import lax
from jax.experimental import pallas as pl
from jax.experimental.pallas import tpu as pltpu
from jax.experimental.pallas import tpu_sc as plsc

NC, NS = 2, 16
NW = NC * NS
K = 128
N_MOL, N_REACT, E, H = 50000, 10000, 320000, 128
NUM_LAYERS = 4

RA = 10112
PER_A = RA // NS
NA = 79
RB = 50048
PER_B = RB // NS
NB = 157
GW = 32
NG = H // GW

_mesh = plsc.VectorSubcoreMesh(
    core_axis_name="c", subcore_axis_name="s", num_cores=NC, num_subcores=NS)


def _zero_rows(buf, nrow, ncol):
    def zb(r, _):
        for q in range(ncol // 16):
            buf[r, pl.ds(q * 16, 16)] = jnp.zeros((16,), jnp.float32)
        return 0
    lax.fori_loop(0, nrow, zb, 0)


def _zero_acc_slice(zbuf, acc, base, nrows):
    nfull, rem = nrows // K, nrows % K
    for t in range(nfull):
        pltpu.sync_copy(zbuf, acc.at[pl.ds(base + t * K, K)])
    if rem:
        pltpu.sync_copy(zbuf.at[pl.ds(0, rem)], acc.at[pl.ds(base + nfull * K, rem)])


def _agg_m2r_body(tbl, src_i, dst_i, out, srcb, dstb, sidx, didx, rows, acc):
    c = lax.axis_index("c")
    s = lax.axis_index("s")
    wid = c * NS + s
    base = s * PER_A
    _zero_rows(rows, K, H)
    _zero_acc_slice(rows, acc, base, PER_A)
    pltpu.sync_copy(src_i.at[wid], srcb)
    pltpu.sync_copy(dst_i.at[wid], dstb)
    plsc.subcore_barrier()

    def body(j, _):
        for q in range(8):
            sidx[pl.ds(q * 16, 16)] = srcb[j, pl.ds(q * 16, 16)]
            didx[pl.ds(q * 16, 16)] = dstb[j, pl.ds(q * 16, 16)]
        pltpu.sync_copy(tbl.at[sidx], rows)
        pltpu.sync_copy(rows, acc.at[didx], add=True)
        return 0
    lax.fori_loop(0, NA, body, 0)
    plsc.subcore_barrier()
    pltpu.sync_copy(acc.at[pl.ds(base, PER_A)], out.at[c, pl.ds(base, PER_A)])


def _agg_r2m_body(pair_i, tbl, out, pairb, sidx, didx, rows, acc):
    c = lax.axis_index("c")
    s = lax.axis_index("s")
    base = s * PER_B
    pltpu.sync_copy(pair_i.at[s], pairb)
    for p in range(2):
        g = c * 2 + p
        _zero_rows(rows, K, GW)
        _zero_acc_slice(rows, acc, base, PER_B)
        plsc.subcore_barrier()
        off = g * N_REACT

        def body(j, _):
            for q in range(8):
                v = pairb[j, pl.ds(q * 16, 16)]
                sidx[pl.ds(q * 16, 16)] = lax.shift_right_logical(v, 16) + off
                didx[pl.ds(q * 16, 16)] = lax.bitwise_and(v, 0xFFFF)
            pltpu.sync_copy(tbl.at[sidx], rows)
            pltpu.sync_copy(rows, acc.at[didx], add=True)
            return 0
        lax.fori_loop(0, NB, body, 0)
        plsc.subcore_barrier()
        pltpu.sync_copy(acc.at[pl.ds(base, PER_B)], out.at[g, pl.ds(base, PER_B)])
        plsc.subcore_barrier()


def _cnt_body(dst_r, dst_m, out_r, out_m, drb, dmb, didx, ones, zbuf,
              acc_r, acc_m):
    c = lax.axis_index("c")
    s = lax.axis_index("s")
    wid = c * NS + s
    _zero_rows(zbuf, K, 16)

    def ob(r, _):
        ones[r, pl.ds(0, 16)] = jnp.ones((16,), jnp.float32)
        return 0
    lax.fori_loop(0, K, ob, 0)
    _zero_acc_slice(zbuf, acc_r, s * PER_A, PER_A)
    _zero_acc_slice(zbuf, acc_m, s * PER_B, PER_B)
    pltpu.sync_copy(dst_r.at[wid], drb)
    pltpu.sync_copy(dst_m.at[wid], dmb)
    plsc.subcore_barrier()

    def body_r(j, _):
        for q in range(8):
            didx[pl.ds(q * 16, 16)] = drb[j, pl.ds(q * 16, 16)]
        pltpu.sync_copy(ones, acc_r.at[didx], add=True)
        return 0
    lax.fori_loop(0, NA, body_r, 0)

    def body_m(j, _):
        for q in range(8):
            didx[pl.ds(q * 16, 16)] = dmb[j, pl.ds(q * 16, 16)]
        pltpu.sync_copy(ones, acc_m.at[didx], add=True)
        return 0
    lax.fori_loop(0, NA, body_m, 0)
    plsc.subcore_barrier()
    pltpu.sync_copy(acc_r.at[pl.ds(s * PER_A, PER_A)],
                    out_r.at[c, pl.ds(s * PER_A, PER_A)])
    pltpu.sync_copy(acc_m.at[pl.ds(s * PER_B, PER_B)],
                    out_m.at[c, pl.ds(s * PER_B, PER_B)])


_kA = pl.kernel(
    _agg_m2r_body,
    out_type=jax.ShapeDtypeStruct((NC, RA, H), jnp.float32),
    mesh=_mesh,
    scratch_types=[
        pltpu.VMEM((NA, K), jnp.int32),
        pltpu.VMEM((NA, K), jnp.int32),
        pltpu.VMEM((K,), jnp.int32),
        pltpu.VMEM((K,), jnp.int32),
        pltpu.VMEM((K, H), jnp.float32),
        pltpu.VMEM_SHARED((RA, H), jnp.float32),
    ],
    compiler_params=pltpu.CompilerParams(use_tc_tiling_on_sc=False),
)

_kB = pl.kernel(
    _agg_r2m_body,
    out_type=jax.ShapeDtypeStruct((NG, RB, GW), jnp.float32),
    mesh=_mesh,
    scratch_types=[
        pltpu.VMEM((NB, K), jnp.int32),
        pltpu.VMEM((K,), jnp.int32),
        pltpu.VMEM((K,), jnp.int32),
        pltpu.VMEM((K, GW), jnp.float32),
        pltpu.VMEM_SHARED((RB, GW), jnp.float32),
    ],
    compiler_params=pltpu.CompilerParams(use_tc_tiling_on_sc=False),
)

_kCNT = pl.kernel(
    _cnt_body,
    out_type=(jax.ShapeDtypeStruct((NC, RA, 16), jnp.float32),
              jax.ShapeDtypeStruct((NC, RB, 16), jnp.float32)),
    mesh=_mesh,
    scratch_types=[
        pltpu.VMEM((NA, K), jnp.int32),
        pltpu.VMEM((NA, K), jnp.int32),
        pltpu.VMEM((K,), jnp.int32),
        pltpu.VMEM((K, 16), jnp.float32),
        pltpu.VMEM((K, 16), jnp.float32),
        pltpu.VMEM_SHARED((RA, 16), jnp.float32),
        pltpu.VMEM_SHARED((RB, 16), jnp.float32),
    ],
    compiler_params=pltpu.CompilerParams(use_tc_tiling_on_sc=False),
)


def _cr_tc(agg_ref, cnt_ref, x_ref, wl_ref, wr_ref, b_ref, o_ref):
    p = agg_ref[...]
    cnt = cnt_ref[...]
    c = cnt[0, :, 0] + cnt[1, :, 0]
    inv = 1.0 / jnp.maximum(c, 1.0)
    mean = (p[0] + p[1]) * inv[:, None]
    o = (jnp.dot(mean, wl_ref[...], preferred_element_type=jnp.float32)
         + jnp.dot(x_ref[...], wr_ref[...], preferred_element_type=jnp.float32)
         + b_ref[...])
    o_ref[...] = jnp.maximum(o, 0.0)


def _cm_tc(agg_ref, cnt_ref, x_ref, wl_ref, wr_ref, b_ref, o_ref):
    a = agg_ref[...]
    cnt = cnt_ref[...]
    c = cnt[0, :, 0] + cnt[1, :, 0]
    inv = 1.0 / jnp.maximum(c, 1.0)
    wl = wl_ref[...]
    acc = (jnp.dot(x_ref[...], wr_ref[...], preferred_element_type=jnp.float32)
           + b_ref[...])
    for g in range(NG):
        acc = acc + jnp.dot(a[g] * inv[:, None], wl[g * GW:(g + 1) * GW, :],
                            preferred_element_type=jnp.float32)
    o_ref[...] = jnp.maximum(acc, 0.0)


def _proj_tc(x_ref, w_ref, b_ref, o_ref):
    o_ref[...] = (jnp.dot(x_ref[...], w_ref[...],
                          preferred_element_type=jnp.float32) + b_ref[...])


_BR = 1000


def _make_cr():
    return pl.pallas_call(
        _cr_tc,
        grid=(N_REACT // _BR,),
        in_specs=[
            pl.BlockSpec((NC, _BR, H), lambda i: (0, i, 0)),
            pl.BlockSpec((NC, _BR, 16), lambda i: (0, i, 0)),
            pl.BlockSpec((_BR, H), lambda i: (i, 0)),
            pl.BlockSpec((H, H), lambda i: (0, 0)),
            pl.BlockSpec((H, H), lambda i: (0, 0)),
            pl.BlockSpec((1, H), lambda i: (0, 0)),
        ],
        out_specs=pl.BlockSpec((_BR, H), lambda i: (i, 0)),
        out_shape=jax.ShapeDtypeStruct((N_REACT, H), jnp.float32),
    )


def _make_cm():
    return pl.pallas_call(
        _cm_tc,
        grid=(N_MOL // _BR,),
        in_specs=[
            pl.BlockSpec((NG, _BR, GW), lambda i: (0, i, 0)),
            pl.BlockSpec((NC, _BR, 16), lambda i: (0, i, 0)),
            pl.BlockSpec((_BR, H), lambda i: (i, 0)),
            pl.BlockSpec((H, H), lambda i: (0, 0)),
            pl.BlockSpec((H, H), lambda i: (0, 0)),
            pl.BlockSpec((1, H), lambda i: (0, 0)),
        ],
        out_specs=pl.BlockSpec((_BR, H), lambda i: (i, 0)),
        out_shape=jax.ShapeDtypeStruct((N_MOL, H), jnp.float32),
    )


def _make_proj():
    return pl.pallas_call(
        _proj_tc,
        grid=(N_REACT // _BR,),
        in_specs=[
            pl.BlockSpec((_BR, H), lambda i: (i, 0)),
            pl.BlockSpec((H, 16), lambda i: (0, 0)),
            pl.BlockSpec((1, 16), lambda i: (0, 0)),
        ],
        out_specs=pl.BlockSpec((_BR, 16), lambda i: (i, 0)),
        out_shape=jax.ShapeDtypeStruct((N_REACT, 16), jnp.float32),
    )


def _tile_chunks(a, nt, nch, pad_val):
    per = E // nt
    a = a.reshape(nt, per)
    a = jnp.pad(a, ((0, 0), (0, nch * K - per)), constant_values=pad_val)
    return a.reshape(nt, nch, K)


def kernel(x_molecule, x_reaction, edge_index_m2r, edge_index_r2m, params):
    srcA = _tile_chunks(edge_index_m2r[0].astype(jnp.int32), NW, NA, 0)
    dstA = _tile_chunks(edge_index_m2r[1].astype(jnp.int32), NW, NA, N_REACT)
    pairB = _tile_chunks(
        (edge_index_r2m[0].astype(jnp.int32) << 16)
        | edge_index_r2m[1].astype(jnp.int32), NS, NB, N_MOL)
    dstMc = _tile_chunks(edge_index_r2m[1].astype(jnp.int32), NW, NA, N_MOL)

    cr = _make_cr()
    cm = _make_cm()
    cnt_r, cnt_m = _kCNT(dstA, dstMc)

    xm, xr = x_molecule, x_reaction
    for l in range(NUM_LAYERS):
        agg_r = _kA(xm, srcA, dstA)
        xr_new = cr(agg_r, cnt_r, xr,
                    params[f"W_l_m2r_{l}"].T, params[f"W_r_m2r_{l}"].T,
                    params[f"b_l_m2r_{l}"].reshape(1, H))
        if l < NUM_LAYERS - 1:
            tblB = xr.reshape(N_REACT, NG, GW).transpose(1, 0, 2)
            tblB = tblB.reshape(NG * N_REACT, GW)
            agg_m = _kB(pairB, tblB)
            xm = cm(agg_m, cnt_m, xm,
                    params[f"W_l_r2m_{l}"].T, params[f"W_r_r2m_{l}"].T,
                    params[f"b_l_r2m_{l}"].reshape(1, H))
        xr = xr_new

    wo = jnp.zeros((H, 16), jnp.float32).at[:, :10].set(params["W_out"].T)
    bo = jnp.zeros((1, 16), jnp.float32).at[0, :10].set(params["b_out"])
    out = _make_proj()(xr, wo, bo)
    return out[:, :10]
